# Initial kernel scaffold; baseline (speedup 1.0000x reference)
import jax
import jax.numpy as jnp
from jax.experimental import pallas as pl


def kernel(x):
    raise NotImplementedError("write your pallas kernel here")


# speedup vs baseline: 14.8525x; 14.8525x over previous
import jax
import jax.numpy as jnp
from jax.experimental import pallas as pl

TOPK_K = 256


def _body(x_ref, o_ref):
    xv = x_ref[...]
    bits = jax.lax.bitcast_convert_type(xv, jnp.int32) & jnp.int32(0x7FFFFFFF)
    br = xv.shape[0]

    def step(i, t):
        cand = t | jax.lax.shift_left(jnp.int32(1), 30 - i)
        cnt = jnp.sum((bits >= cand).astype(jnp.int32), axis=1, keepdims=True)
        return jnp.where(cnt >= TOPK_K, cand, t)

    t = jax.lax.fori_loop(0, 31, step, jnp.zeros((br, 1), jnp.int32))
    o_ref[...] = jnp.where(bits >= t, xv, 0.0)


def kernel(x):
    rows, n = x.shape
    br = 8
    return pl.pallas_call(
        _body,
        grid=(rows // br,),
        in_specs=[pl.BlockSpec((br, n), lambda i: (i, 0))],
        out_specs=pl.BlockSpec((br, n), lambda i: (i, 0)),
        out_shape=jax.ShapeDtypeStruct(x.shape, x.dtype),
    )(x)
